# baseline (speedup 1.0000x reference)
import functools

import jax
import jax.numpy as jnp
from jax import lax
from jax.experimental import pallas as pl
from jax.experimental.pallas import tpu as pltpu
from jax.experimental.pallas import tpu_sc as plsc

N = 10000
E = 320000
D = 128
XP = 4
CW = 2 * D
DEG = 32.0

NC = 2
NS = 16
NW = NC * NS
EPW = E // NW
K = 80
NG = K // 16
NCH = EPW // K

NP = 10240
CR = NP // NW
ACC_R = CR + 1
KS = 64
CAP = 10688
SCN = 1600
NSC = E // SCN
IDB = 19

f32 = jnp.float32


def _prep_body(h_ref, w1a_ref, w1b_ref, b1_ref, p_ref, q_ref):
    hh = h_ref[...]
    p_ref[...] = jnp.dot(hh, w1a_ref[...], preferred_element_type=f32) + b1_ref[...]
    q_ref[...] = jnp.dot(hh, w1b_ref[...], preferred_element_type=f32)


def _prep(h, w1a, w1b, b1):
    bn = 2000
    return pl.pallas_call(
        _prep_body,
        grid=(N // bn,),
        in_specs=[
            pl.BlockSpec((bn, D), lambda i: (i, 0)),
            pl.BlockSpec((D, D), lambda i: (0, 0)),
            pl.BlockSpec((D, D), lambda i: (0, 0)),
            pl.BlockSpec((1, D), lambda i: (0, 0)),
        ],
        out_specs=[
            pl.BlockSpec((bn, D), lambda i: (i, 0)),
            pl.BlockSpec((bn, D), lambda i: (i, 0)),
        ],
        out_shape=[
            jax.ShapeDtypeStruct((N, D), f32),
            jax.ShapeDtypeStruct((N, D), f32),
        ],
    )(h, w1a, w1b, b1)


@functools.lru_cache(maxsize=None)
def _sc_kernels():
    mesh = plsc.VectorSubcoreMesh(core_axis_name="c", subcore_axis_name="s")
    iota16 = lambda: lax.iota(jnp.int32, 16)
    params = pltpu.CompilerParams(needs_layout_passes=False)

    @functools.partial(
        pl.kernel,
        mesh=mesh,
        compiler_params=params,
        out_type=[
            jax.ShapeDtypeStruct((E, D), f32),
            jax.ShapeDtypeStruct((E, D), f32),
            jax.ShapeDtypeStruct((E * XP,), f32),
        ],
        scratch_types=[
            pltpu.VMEM((EPW,), jnp.int32),
            pltpu.VMEM((EPW,), jnp.int32),
            pltpu.VMEM((K, D), f32),
            pltpu.VMEM((K, D), f32),
            pltpu.VMEM((K, D), f32),
            pltpu.VMEM((K, D), f32),
            pltpu.VMEM((K * XP,), f32),
            pltpu.VMEM((K * XP,), f32),
            pltpu.VMEM((N * XP,), f32),
            pltpu.SemaphoreType.DMA,
            pltpu.SemaphoreType.DMA,
            pltpu.SemaphoreType.DMA,
        ],
    )
    def sc_gather(ptab, qtab, xtab, dst, src,
                  pd_o, qs_o, dd_o,
                  dall, sall, pbuf0, pbuf1, qbuf0, qbuf1, dbuf0, dbuf1,
                  xloc, semA, semB, wsem):
        wid = lax.axis_index("s") * NC + lax.axis_index("c")
        base = wid * EPW
        pltpu.sync_copy(xtab, xloc)
        pltpu.sync_copy(dst.at[pl.ds(base, EPW)], dall)
        pltpu.sync_copy(src.at[pl.ds(base, EPW)], sall)

        def xcompute(ci, dbuf):
            for g in range(NG):
                i16d = dall[pl.ds(ci * K + g * 16, 16)] * XP
                i16s = sall[pl.ds(ci * K + g * 16, 16)] * XP
                rows = (g * 16 + iota16()) * XP
                diffs = []
                for comp in range(3):
                    xd = plsc.load_gather(xloc, [i16d + comp])
                    xs = plsc.load_gather(xloc, [i16s + comp])
                    dc = xd - xs
                    diffs.append(dc)
                    plsc.store_scatter(dbuf, [rows + comp], dc)
                d2 = (diffs[0] * diffs[0] + diffs[1] * diffs[1]
                      + diffs[2] * diffs[2])
                plsc.store_scatter(dbuf, [rows + 3], d2)

        def do_chunk(ci, pbuf, qbuf, sem):
            off = base + ci * K
            c1 = pltpu.async_copy(ptab.at[dall.at[pl.ds(ci * K, K)]], pbuf, sem)
            c2 = pltpu.async_copy(qtab.at[sall.at[pl.ds(ci * K, K)]], qbuf, sem)
            return off, c1, c2

        def writes(off, pbuf, qbuf, dbuf):
            w1 = pltpu.async_copy(pbuf, pd_o.at[pl.ds(off, K)], wsem)
            w2 = pltpu.async_copy(qbuf, qs_o.at[pl.ds(off, K)], wsem)
            w3 = pltpu.async_copy(dbuf, dd_o.at[pl.ds(off * XP, K * XP)], wsem)
            return w1, w2, w3

        def pair_body(a, carry):
            off0, c1, c2 = do_chunk(2 * a, pbuf0, qbuf0, semA)
            off1, c3, c4 = do_chunk(2 * a + 1, pbuf1, qbuf1, semB)
            xcompute(2 * a, dbuf0)
            c1.wait(); c2.wait()
            ws0 = writes(off0, pbuf0, qbuf0, dbuf0)
            xcompute(2 * a + 1, dbuf1)
            c3.wait(); c4.wait()
            ws1 = writes(off1, pbuf1, qbuf1, dbuf1)
            for w in (*ws0, *ws1):
                w.wait()
            return carry

        lax.fori_loop(0, NCH // 2, pair_body, 0)
        off0, c1, c2 = do_chunk(NCH - 1, pbuf0, qbuf0, semA)
        xcompute(NCH - 1, dbuf0)
        c1.wait(); c2.wait()
        for w in writes(off0, pbuf0, qbuf0, dbuf0):
            w.wait()

    @functools.partial(
        pl.kernel,
        mesh=mesh,
        compiler_params=params,
        out_type=jax.ShapeDtypeStruct((NP * CW,), f32),
        scratch_types=[
            pltpu.VMEM((SCN,), jnp.int32),
            pltpu.VMEM((SCN,), jnp.int32),
            pltpu.VMEM((CAP + 2 * KS,), jnp.int32),
            pltpu.VMEM((KS,), jnp.int32),
            pltpu.VMEM((KS,), jnp.int32),
            pltpu.VMEM((KS,), jnp.int32),
            pltpu.VMEM((KS,), jnp.int32),
            pltpu.VMEM((KS, CW), f32),
            pltpu.VMEM((KS, CW), f32),
            pltpu.VMEM((ACC_R * CW,), f32),
            pltpu.SemaphoreType.DMA,
            pltpu.SemaphoreType.DMA,
        ],
    )
    def sc_scatter(comb, dst, zacc,
                   agg_o,
                   scan0, scan1, packed, ids0, ids1, lidx0, lidx1,
                   rb0, rb1, acc, semA, semB):
        c = lax.axis_index("c")
        s = lax.axis_index("s")
        wid = s * NC + c
        lo = wid * CR
        hi = lo + CR

        pltpu.sync_copy(zacc, acc)

        def process(scanbuf, ch, cnt):
            def vreg_body(g, cnt):
                for gg in (4 * g, 4 * g + 1, 4 * g + 2, 4 * g + 3):
                    v = scanbuf[pl.ds(gg * 16, 16)]
                    mask = (v >= lo) & (v < hi)
                    gid = ch * SCN + gg * 16 + iota16()
                    pk = gid | ((v - lo) << IDB)
                    plsc.store_compressed(packed.at[pl.ds(cnt, 16)], pk,
                                          mask=mask)
                    nm = plsc.all_reduce_population_count(mask)
                    cnt = cnt + nm[0]
                return cnt

            return lax.fori_loop(0, SCN // 64, vreg_body, cnt)

        def scan_pair(a, cnt):
            c0 = pltpu.async_copy(dst.at[pl.ds((2 * a) * SCN, SCN)],
                                  scan0, semA)
            c1 = pltpu.async_copy(dst.at[pl.ds((2 * a + 1) * SCN, SCN)],
                                  scan1, semB)
            c0.wait()
            cnt = process(scan0, 2 * a, cnt)
            c1.wait()
            cnt = process(scan1, 2 * a + 1, cnt)
            return cnt

        cnt = lax.fori_loop(0, NSC // 2, scan_pair, 0)

        for r in range(2 * KS // 16):
            packed[pl.ds(cnt + r * 16, 16)] = jnp.full((16,), CR << IDB,
                                                       jnp.int32)
        npairs = (cnt + 2 * KS - 1) >> 7

        def unpack(j, idsbuf, lidxbuf):
            for r in range(KS // 16):
                pv = packed[pl.ds(j * KS + r * 16, 16)]
                idsbuf[pl.ds(r * 16, 16)] = pv & ((1 << IDB) - 1)
                lidxbuf[pl.ds(r * 16, 16)] = pv >> IDB

        def accum(rowbuf, lidxbuf):
            def row_body(i2, carry2):
                for i in (2 * i2, 2 * i2 + 1):
                    lsplat = plsc.load_gather(
                        lidxbuf,
                        [jnp.broadcast_to(i, (16,)).astype(jnp.int32)])
                    base = lsplat * CW + iota16()
                    for jj in range((D + XP + 15) // 16):
                        v = rowbuf[i, pl.ds(jj * 16, 16)]
                        plsc.addupdate_scatter(acc, [base + jj * 16], v)
                return carry2

            lax.fori_loop(0, KS // 2, row_body, 0)

        def pair_body(a, carry):
            unpack(2 * a, ids0, lidx0)
            g0 = pltpu.async_copy(comb.at[ids0], rb0, semA)
            unpack(2 * a + 1, ids1, lidx1)
            g1 = pltpu.async_copy(comb.at[ids1], rb1, semB)
            g0.wait()
            accum(rb0, lidx0)
            g1.wait()
            accum(rb1, lidx1)
            return carry

        lax.fori_loop(0, npairs, pair_body, 0)

        pltpu.sync_copy(acc.at[pl.ds(0, CR * CW)],
                        agg_o.at[pl.ds(lo * CW, CR * CW)])

    return sc_gather, sc_scatter


def _edge_body(pd_ref, qs_ref, dd_ref, w2_ref, b2_ref, wc_ref,
               w1c_ref, comb_ref):
    dd = dd_ref[...]
    d2 = dd[:, 3:4]
    t1 = pd_ref[...] + qs_ref[...] + d2 * w1c_ref[...]
    a1 = t1 * jax.nn.sigmoid(t1)
    t2 = jnp.dot(a1, w2_ref[...], preferred_element_type=f32) + b2_ref[...]
    m = t2 * jax.nn.sigmoid(t2)
    cw = jnp.sum(m * wc_ref[...], axis=1, keepdims=True)
    comb_ref[:, :D] = m
    comb_ref[:, D:D + XP] = dd * cw


def _edge(pd, qs, dd, w2, b2, wc_row, w1c):
    be = 2000
    return pl.pallas_call(
        _edge_body,
        grid=(E // be,),
        in_specs=[
            pl.BlockSpec((be, D), lambda i: (i, 0)),
            pl.BlockSpec((be, D), lambda i: (i, 0)),
            pl.BlockSpec((be, XP), lambda i: (i, 0)),
            pl.BlockSpec((D, D), lambda i: (0, 0)),
            pl.BlockSpec((1, D), lambda i: (0, 0)),
            pl.BlockSpec((1, D), lambda i: (0, 0)),
            pl.BlockSpec((1, D), lambda i: (0, 0)),
        ],
        out_specs=pl.BlockSpec((be, CW), lambda i: (i, 0)),
        out_shape=jax.ShapeDtypeStruct((E, CW), f32),
    )(pd, qs, dd, w2, b2, wc_row, w1c)


def _node_body(h_ref, xp_ref, agg_ref, wu1a_ref, wu1b_ref, bu1_ref,
               wu2_ref, bu2_ref, xo_ref, ho_ref):
    agg = agg_ref[:, :D]
    aggx = agg_ref[:, D:D + XP]
    t = (jnp.dot(h_ref[...], wu1a_ref[...], preferred_element_type=f32)
         + jnp.dot(agg, wu1b_ref[...], preferred_element_type=f32)
         + bu1_ref[...])
    a = t * jax.nn.sigmoid(t)
    ho_ref[...] = (h_ref[...]
                   + jnp.dot(a, wu2_ref[...], preferred_element_type=f32)
                   + bu2_ref[...])
    xo_ref[...] = xp_ref[...] + aggx * (1.0 / DEG)


def _node(h, xp, agg, wu1a, wu1b, bu1, wu2, bu2):
    bn = 2000
    return pl.pallas_call(
        _node_body,
        grid=(N // bn,),
        in_specs=[
            pl.BlockSpec((bn, D), lambda i: (i, 0)),
            pl.BlockSpec((bn, XP), lambda i: (i, 0)),
            pl.BlockSpec((bn, CW), lambda i: (i, 0)),
            pl.BlockSpec((D, D), lambda i: (0, 0)),
            pl.BlockSpec((D, D), lambda i: (0, 0)),
            pl.BlockSpec((1, D), lambda i: (0, 0)),
            pl.BlockSpec((D, D), lambda i: (0, 0)),
            pl.BlockSpec((1, D), lambda i: (0, 0)),
        ],
        out_specs=[
            pl.BlockSpec((bn, XP), lambda i: (i, 0)),
            pl.BlockSpec((bn, D), lambda i: (i, 0)),
        ],
        out_shape=[
            jax.ShapeDtypeStruct((N, XP), f32),
            jax.ShapeDtypeStruct((N, D), f32),
        ],
    )(h, xp, agg, wu1a, wu1b, bu1, wu2, bu2)


def kernel(x, h, edge_index, W1, b1, W2, b2, Wc, Wu1, bu1, Wu2, bu2):
    src = edge_index[0].astype(jnp.int32)
    dst = edge_index[1].astype(jnp.int32)
    xp = jnp.pad(x.astype(f32), ((0, 0), (0, XP - x.shape[1])))
    w1a = W1[:D]
    w1b = W1[D:2 * D]
    w1c = W1[2 * D:2 * D + 1]
    b1r = b1.reshape(1, D)
    b2r = b2.reshape(1, D)
    wc_row = Wc.reshape(1, D)
    wu1a = Wu1[:D]
    wu1b = Wu1[D:2 * D]
    bu1r = bu1.reshape(1, D)
    bu2r = bu2.reshape(1, D)

    sc_gather, sc_scatter = _sc_kernels()
    p, q = _prep(h, w1a, w1b, b1r)
    pd, qs, dd = sc_gather(p, q, xp.reshape(-1), dst, src)
    dd = dd.reshape(E, XP)
    comb = _edge(pd, qs, dd, W2, b2r, wc_row, w1c)
    zacc = jnp.zeros((ACC_R * CW,), f32)
    agg = sc_scatter(comb, dst, zacc).reshape(NP, CW)
    xo, ho = _node(h, xp, agg[:N], wu1a, wu1b, bu1r, Wu2, bu2r)
    return (xo[:, :x.shape[1]], ho)


# speedup vs baseline: 1.1941x; 1.1941x over previous
import functools

import jax
import jax.numpy as jnp
from jax import lax
from jax.experimental import pallas as pl
from jax.experimental.pallas import tpu as pltpu
from jax.experimental.pallas import tpu_sc as plsc

N = 10000
E = 320000
D = 128
XP = 4
CW = 2 * D
DEG = 32.0

NC = 2
NS = 16
NW = NC * NS
EPW = E // NW
K = 80
NG = K // 16
NCH = EPW // K

NP = 10240
CR = NP // NW
ACC_R = CR + 1
KS = 64
CAPP = 10880
SCN = 1600
NSC = E // SCN
IDB = 19

f32 = jnp.float32


def _prep_body(h_ref, w1a_ref, w1b_ref, b1_ref, p_ref, q_ref):
    hh = h_ref[...]
    p_ref[...] = jnp.dot(hh, w1a_ref[...], preferred_element_type=f32) + b1_ref[...]
    q_ref[...] = jnp.dot(hh, w1b_ref[...], preferred_element_type=f32)


def _prep(h, w1a, w1b, b1):
    bn = 2000
    return pl.pallas_call(
        _prep_body,
        grid=(N // bn,),
        in_specs=[
            pl.BlockSpec((bn, D), lambda i: (i, 0)),
            pl.BlockSpec((D, D), lambda i: (0, 0)),
            pl.BlockSpec((D, D), lambda i: (0, 0)),
            pl.BlockSpec((1, D), lambda i: (0, 0)),
        ],
        out_specs=[
            pl.BlockSpec((bn, D), lambda i: (i, 0)),
            pl.BlockSpec((bn, D), lambda i: (i, 0)),
        ],
        out_shape=[
            jax.ShapeDtypeStruct((N, D), f32),
            jax.ShapeDtypeStruct((N, D), f32),
        ],
    )(h, w1a, w1b, b1)


@functools.lru_cache(maxsize=None)
def _sc_kernels():
    mesh = plsc.VectorSubcoreMesh(core_axis_name="c", subcore_axis_name="s")
    iota16 = lambda: lax.iota(jnp.int32, 16)
    params = pltpu.CompilerParams(needs_layout_passes=False)

    @functools.partial(
        pl.kernel,
        mesh=mesh,
        compiler_params=params,
        out_type=[
            jax.ShapeDtypeStruct((E, D), f32),
            jax.ShapeDtypeStruct((E, D), f32),
            jax.ShapeDtypeStruct((E * XP,), f32),
        ],
        scratch_types=[
            pltpu.VMEM((EPW,), jnp.int32),
            pltpu.VMEM((EPW,), jnp.int32),
            pltpu.VMEM((K, D), f32),
            pltpu.VMEM((K, D), f32),
            pltpu.VMEM((K, D), f32),
            pltpu.VMEM((K, D), f32),
            pltpu.VMEM((K * XP,), f32),
            pltpu.VMEM((K * XP,), f32),
            pltpu.VMEM((N * XP,), f32),
            pltpu.SemaphoreType.DMA,
            pltpu.SemaphoreType.DMA,
            pltpu.SemaphoreType.DMA,
        ],
    )
    def sc_gather(ptab, qtab, xtab, dst, src,
                  pd_o, qs_o, dd_o,
                  dall, sall, pbuf0, pbuf1, qbuf0, qbuf1, dbuf0, dbuf1,
                  xloc, semA, semB, wsem):
        wid = lax.axis_index("s") * NC + lax.axis_index("c")
        base = wid * EPW
        pltpu.sync_copy(xtab, xloc)
        pltpu.sync_copy(dst.at[pl.ds(base, EPW)], dall)
        pltpu.sync_copy(src.at[pl.ds(base, EPW)], sall)

        def xcompute(ci, dbuf):
            for g in range(NG):
                i16d = dall[pl.ds(ci * K + g * 16, 16)] * XP
                i16s = sall[pl.ds(ci * K + g * 16, 16)] * XP
                rows = (g * 16 + iota16()) * XP
                diffs = []
                for comp in range(3):
                    xd = plsc.load_gather(xloc, [i16d + comp])
                    xs = plsc.load_gather(xloc, [i16s + comp])
                    dc = xd - xs
                    diffs.append(dc)
                    plsc.store_scatter(dbuf, [rows + comp], dc)
                d2 = (diffs[0] * diffs[0] + diffs[1] * diffs[1]
                      + diffs[2] * diffs[2])
                plsc.store_scatter(dbuf, [rows + 3], d2)

        def do_chunk(ci, pbuf, qbuf, sem):
            off = base + ci * K
            c1 = pltpu.async_copy(ptab.at[dall.at[pl.ds(ci * K, K)]], pbuf, sem)
            c2 = pltpu.async_copy(qtab.at[sall.at[pl.ds(ci * K, K)]], qbuf, sem)
            return off, c1, c2

        def writes(off, pbuf, qbuf, dbuf):
            w1 = pltpu.async_copy(pbuf, pd_o.at[pl.ds(off, K)], wsem)
            w2 = pltpu.async_copy(qbuf, qs_o.at[pl.ds(off, K)], wsem)
            w3 = pltpu.async_copy(dbuf, dd_o.at[pl.ds(off * XP, K * XP)], wsem)
            return w1, w2, w3

        def pair_body(a, carry):
            off0, c1, c2 = do_chunk(2 * a, pbuf0, qbuf0, semA)
            off1, c3, c4 = do_chunk(2 * a + 1, pbuf1, qbuf1, semB)
            xcompute(2 * a, dbuf0)
            c1.wait(); c2.wait()
            ws0 = writes(off0, pbuf0, qbuf0, dbuf0)
            xcompute(2 * a + 1, dbuf1)
            c3.wait(); c4.wait()
            ws1 = writes(off1, pbuf1, qbuf1, dbuf1)
            for w in (*ws0, *ws1):
                w.wait()
            return carry

        lax.fori_loop(0, NCH // 2, pair_body, 0)
        off0, c1, c2 = do_chunk(NCH - 1, pbuf0, qbuf0, semA)
        xcompute(NCH - 1, dbuf0)
        c1.wait(); c2.wait()
        for w in writes(off0, pbuf0, qbuf0, dbuf0):
            w.wait()

    @functools.partial(
        pl.kernel,
        mesh=mesh,
        compiler_params=params,
        out_type=jax.ShapeDtypeStruct((NW, CAPP), jnp.int32),
        scratch_types=[
            pltpu.VMEM((SCN,), jnp.int32),
            pltpu.VMEM((SCN,), jnp.int32),
            pltpu.VMEM((CAPP,), jnp.int32),
            pltpu.SemaphoreType.DMA,
            pltpu.SemaphoreType.DMA,
        ],
    )
    def sc_scan(dst,
                packed_o,
                scan0, scan1, packed, semA, semB):
        c = lax.axis_index("c")
        s = lax.axis_index("s")
        wid = s * NC + c
        lo = wid * CR
        hi = lo + CR

        def process(scanbuf, ch, cnt):
            def vreg_body(g, cnt):
                for gg in (4 * g, 4 * g + 1, 4 * g + 2, 4 * g + 3):
                    v = scanbuf[pl.ds(gg * 16, 16)]
                    mask = (v >= lo) & (v < hi)
                    gid = ch * SCN + gg * 16 + iota16()
                    pk = gid | ((v - lo) << IDB)
                    plsc.store_compressed(packed.at[pl.ds(cnt, 16)], pk,
                                          mask=mask)
                    nm = plsc.all_reduce_population_count(mask)
                    cnt = cnt + nm[0]
                return cnt

            return lax.fori_loop(0, SCN // 64, vreg_body, cnt)

        def scan_pair(a, cnt):
            c0 = pltpu.async_copy(dst.at[pl.ds((2 * a) * SCN, SCN)],
                                  scan0, semA)
            c1 = pltpu.async_copy(dst.at[pl.ds((2 * a + 1) * SCN, SCN)],
                                  scan1, semB)
            c0.wait()
            cnt = process(scan0, 2 * a, cnt)
            c1.wait()
            cnt = process(scan1, 2 * a + 1, cnt)
            return cnt

        cnt = lax.fori_loop(0, NSC // 2, scan_pair, 0)

        def pad_body(g, carry):
            slot = g * 16 + iota16()
            pk = slot | (CR << IDB)
            plsc.store_scatter(packed, [slot], pk, mask=slot >= cnt)
            return carry

        lax.fori_loop(0, CAPP // 16, pad_body, 0)
        pltpu.sync_copy(packed, packed_o.at[wid])

    @functools.partial(
        pl.kernel,
        mesh=mesh,
        compiler_params=params,
        out_type=jax.ShapeDtypeStruct((NP * CW,), f32),
        scratch_types=[
            pltpu.VMEM((CAPP,), jnp.int32),
            pltpu.VMEM((KS,), jnp.int32),
            pltpu.VMEM((KS,), jnp.int32),
            pltpu.VMEM((KS,), jnp.int32),
            pltpu.VMEM((KS,), jnp.int32),
            pltpu.VMEM((KS, CW), f32),
            pltpu.VMEM((KS, CW), f32),
            pltpu.VMEM((ACC_R * CW,), f32),
            pltpu.SemaphoreType.DMA,
            pltpu.SemaphoreType.DMA,
        ],
    )
    def sc_scatter(comb, packed_i, zacc,
                   agg_o,
                   packed, ids0, ids1, lidx0, lidx1,
                   rb0, rb1, acc, semA, semB):
        c = lax.axis_index("c")
        s = lax.axis_index("s")
        wid = s * NC + c
        lo = wid * CR

        z0 = pltpu.async_copy(zacc, acc, semB)
        pltpu.sync_copy(packed_i.at[wid], packed)
        npairs = CAPP // (2 * KS)
        z0.wait()

        def unpack(j, idsbuf, lidxbuf):
            for r in range(KS // 16):
                pv = packed[pl.ds(j * KS + r * 16, 16)]
                idsbuf[pl.ds(r * 16, 16)] = pv & ((1 << IDB) - 1)
                lidxbuf[pl.ds(r * 16, 16)] = pv >> IDB

        def accum(rowbuf, lidxbuf):
            def row_body(i2, carry2):
                for i in (2 * i2, 2 * i2 + 1):
                    lsplat = plsc.load_gather(
                        lidxbuf,
                        [jnp.broadcast_to(i, (16,)).astype(jnp.int32)])
                    base = lsplat * CW + iota16()
                    for jj in range((D + XP + 15) // 16):
                        v = rowbuf[i, pl.ds(jj * 16, 16)]
                        plsc.addupdate_scatter(acc, [base + jj * 16], v)
                return carry2

            lax.fori_loop(0, KS // 2, row_body, 0)

        def pair_body(a, carry):
            unpack(2 * a, ids0, lidx0)
            g0 = pltpu.async_copy(comb.at[ids0], rb0, semA)
            unpack(2 * a + 1, ids1, lidx1)
            g1 = pltpu.async_copy(comb.at[ids1], rb1, semB)
            g0.wait()
            accum(rb0, lidx0)
            g1.wait()
            accum(rb1, lidx1)
            return carry

        lax.fori_loop(0, npairs, pair_body, 0)

        pltpu.sync_copy(acc.at[pl.ds(0, CR * CW)],
                        agg_o.at[pl.ds(lo * CW, CR * CW)])

    return sc_gather, sc_scan, sc_scatter


def _edge_body(pd_ref, qs_ref, dd_ref, w2_ref, b2_ref, wc_ref,
               w1c_ref, comb_ref):
    dd = dd_ref[...]
    d2 = dd[:, 3:4]
    t1 = pd_ref[...] + qs_ref[...] + d2 * w1c_ref[...]
    a1 = t1 * jax.nn.sigmoid(t1)
    t2 = jnp.dot(a1, w2_ref[...], preferred_element_type=f32) + b2_ref[...]
    m = t2 * jax.nn.sigmoid(t2)
    cw = jnp.sum(m * wc_ref[...], axis=1, keepdims=True)
    comb_ref[:, :D] = m
    comb_ref[:, D:D + XP] = dd * cw


def _edge(pd, qs, dd, w2, b2, wc_row, w1c):
    be = 2000
    return pl.pallas_call(
        _edge_body,
        grid=(E // be,),
        in_specs=[
            pl.BlockSpec((be, D), lambda i: (i, 0)),
            pl.BlockSpec((be, D), lambda i: (i, 0)),
            pl.BlockSpec((be, XP), lambda i: (i, 0)),
            pl.BlockSpec((D, D), lambda i: (0, 0)),
            pl.BlockSpec((1, D), lambda i: (0, 0)),
            pl.BlockSpec((1, D), lambda i: (0, 0)),
            pl.BlockSpec((1, D), lambda i: (0, 0)),
        ],
        out_specs=pl.BlockSpec((be, CW), lambda i: (i, 0)),
        out_shape=jax.ShapeDtypeStruct((E, CW), f32),
    )(pd, qs, dd, w2, b2, wc_row, w1c)


def _node_body(h_ref, xp_ref, agg_ref, wu1a_ref, wu1b_ref, bu1_ref,
               wu2_ref, bu2_ref, xo_ref, ho_ref):
    agg = agg_ref[:, :D]
    aggx = agg_ref[:, D:D + XP]
    t = (jnp.dot(h_ref[...], wu1a_ref[...], preferred_element_type=f32)
         + jnp.dot(agg, wu1b_ref[...], preferred_element_type=f32)
         + bu1_ref[...])
    a = t * jax.nn.sigmoid(t)
    ho_ref[...] = (h_ref[...]
                   + jnp.dot(a, wu2_ref[...], preferred_element_type=f32)
                   + bu2_ref[...])
    xo_ref[...] = xp_ref[...] + aggx * (1.0 / DEG)


def _node(h, xp, agg, wu1a, wu1b, bu1, wu2, bu2):
    bn = 2000
    return pl.pallas_call(
        _node_body,
        grid=(N // bn,),
        in_specs=[
            pl.BlockSpec((bn, D), lambda i: (i, 0)),
            pl.BlockSpec((bn, XP), lambda i: (i, 0)),
            pl.BlockSpec((bn, CW), lambda i: (i, 0)),
            pl.BlockSpec((D, D), lambda i: (0, 0)),
            pl.BlockSpec((D, D), lambda i: (0, 0)),
            pl.BlockSpec((1, D), lambda i: (0, 0)),
            pl.BlockSpec((D, D), lambda i: (0, 0)),
            pl.BlockSpec((1, D), lambda i: (0, 0)),
        ],
        out_specs=[
            pl.BlockSpec((bn, XP), lambda i: (i, 0)),
            pl.BlockSpec((bn, D), lambda i: (i, 0)),
        ],
        out_shape=[
            jax.ShapeDtypeStruct((N, XP), f32),
            jax.ShapeDtypeStruct((N, D), f32),
        ],
    )(h, xp, agg, wu1a, wu1b, bu1, wu2, bu2)


def kernel(x, h, edge_index, W1, b1, W2, b2, Wc, Wu1, bu1, Wu2, bu2):
    src = edge_index[0].astype(jnp.int32)
    dst = edge_index[1].astype(jnp.int32)
    xp = jnp.pad(x.astype(f32), ((0, 0), (0, XP - x.shape[1])))
    w1a = W1[:D]
    w1b = W1[D:2 * D]
    w1c = W1[2 * D:2 * D + 1]
    b1r = b1.reshape(1, D)
    b2r = b2.reshape(1, D)
    wc_row = Wc.reshape(1, D)
    wu1a = Wu1[:D]
    wu1b = Wu1[D:2 * D]
    bu1r = bu1.reshape(1, D)
    bu2r = bu2.reshape(1, D)

    sc_gather, sc_scan, sc_scatter = _sc_kernels()
    p, q = _prep(h, w1a, w1b, b1r)
    packed = sc_scan(dst)
    pd, qs, dd = sc_gather(p, q, xp.reshape(-1), dst, src)
    dd = dd.reshape(E, XP)
    comb = _edge(pd, qs, dd, W2, b2r, wc_row, w1c)
    zacc = jnp.zeros((ACC_R * CW,), f32)
    agg = sc_scatter(comb, packed, zacc).reshape(NP, CW)
    xo, ho = _node(h, xp, agg[:N], wu1a, wu1b, bu1r, Wu2, bu2r)
    return (xo[:, :x.shape[1]], ho)


# speedup vs baseline: 1.1996x; 1.0046x over previous
import functools

import jax
import jax.numpy as jnp
from jax import lax
from jax.experimental import pallas as pl
from jax.experimental.pallas import tpu as pltpu
from jax.experimental.pallas import tpu_sc as plsc

N = 10000
E = 320000
D = 128
XP = 4
CW = 2 * D
DEG = 32.0

NC = 2
NS = 16
NW = NC * NS
EPW = E // NW
K = 80
NG = K // 16
NCH = EPW // K

NP = 10240
CR = NP // NW
ACC_R = CR + 1
KS = 64
CAPP = 10880
SCN = 1600
NSC = E // SCN
IDB = 19

f32 = jnp.float32


def _prep_body(h_ref, w1a_ref, w1b_ref, b1_ref, p_ref, q_ref):
    hh = h_ref[...]
    p_ref[...] = jnp.dot(hh, w1a_ref[...], preferred_element_type=f32) + b1_ref[...]
    q_ref[...] = jnp.dot(hh, w1b_ref[...], preferred_element_type=f32)


def _prep(h, w1a, w1b, b1):
    bn = 2000
    return pl.pallas_call(
        _prep_body,
        grid=(N // bn,),
        in_specs=[
            pl.BlockSpec((bn, D), lambda i: (i, 0)),
            pl.BlockSpec((D, D), lambda i: (0, 0)),
            pl.BlockSpec((D, D), lambda i: (0, 0)),
            pl.BlockSpec((1, D), lambda i: (0, 0)),
        ],
        out_specs=[
            pl.BlockSpec((bn, D), lambda i: (i, 0)),
            pl.BlockSpec((bn, D), lambda i: (i, 0)),
        ],
        out_shape=[
            jax.ShapeDtypeStruct((N, D), f32),
            jax.ShapeDtypeStruct((N, D), f32),
        ],
    )(h, w1a, w1b, b1)


@functools.lru_cache(maxsize=None)
def _sc_kernels():
    mesh = plsc.VectorSubcoreMesh(core_axis_name="c", subcore_axis_name="s")
    iota16 = lambda: lax.iota(jnp.int32, 16)
    params = pltpu.CompilerParams(needs_layout_passes=False)

    @functools.partial(
        pl.kernel,
        mesh=mesh,
        compiler_params=params,
        out_type=[
            jax.ShapeDtypeStruct((E, D), f32),
            jax.ShapeDtypeStruct((E, D), f32),
            jax.ShapeDtypeStruct((E * XP,), f32),
        ],
        scratch_types=[
            pltpu.VMEM((EPW,), jnp.int32),
            pltpu.VMEM((EPW,), jnp.int32),
            pltpu.VMEM((K, D), f32),
            pltpu.VMEM((K, D), f32),
            pltpu.VMEM((K, D), f32),
            pltpu.VMEM((K, D), f32),
            pltpu.VMEM((K * XP,), f32),
            pltpu.VMEM((K * XP,), f32),
            pltpu.VMEM((N * XP,), f32),
            pltpu.SemaphoreType.DMA,
            pltpu.SemaphoreType.DMA,
            pltpu.SemaphoreType.DMA,
        ],
    )
    def sc_gather(ptab, qtab, xtab, dst, src,
                  pd_o, qs_o, dd_o,
                  dall, sall, pbuf0, pbuf1, qbuf0, qbuf1, dbuf0, dbuf1,
                  xloc, semA, semB, wsem):
        wid = lax.axis_index("s") * NC + lax.axis_index("c")
        base = wid * EPW
        pltpu.sync_copy(xtab, xloc)
        pltpu.sync_copy(dst.at[pl.ds(base, EPW)], dall)
        pltpu.sync_copy(src.at[pl.ds(base, EPW)], sall)

        def xcompute(ci, dbuf):
            for g in range(NG):
                i16d = dall[pl.ds(ci * K + g * 16, 16)] * XP
                i16s = sall[pl.ds(ci * K + g * 16, 16)] * XP
                rows = (g * 16 + iota16()) * XP
                diffs = []
                for comp in range(3):
                    xd = plsc.load_gather(xloc, [i16d + comp])
                    xs = plsc.load_gather(xloc, [i16s + comp])
                    dc = xd - xs
                    diffs.append(dc)
                    plsc.store_scatter(dbuf, [rows + comp], dc)
                d2 = (diffs[0] * diffs[0] + diffs[1] * diffs[1]
                      + diffs[2] * diffs[2])
                plsc.store_scatter(dbuf, [rows + 3], d2)

        def do_chunk(ci, pbuf, qbuf, sem):
            off = base + ci * K
            c1 = pltpu.async_copy(ptab.at[dall.at[pl.ds(ci * K, K)]], pbuf, sem)
            c2 = pltpu.async_copy(qtab.at[sall.at[pl.ds(ci * K, K)]], qbuf, sem)
            return off, c1, c2

        def writes(off, pbuf, qbuf, dbuf):
            w1 = pltpu.async_copy(pbuf, pd_o.at[pl.ds(off, K)], wsem)
            w2 = pltpu.async_copy(qbuf, qs_o.at[pl.ds(off, K)], wsem)
            w3 = pltpu.async_copy(dbuf, dd_o.at[pl.ds(off * XP, K * XP)], wsem)
            return w1, w2, w3

        def pair_body(a, carry):
            off0, c1, c2 = do_chunk(2 * a, pbuf0, qbuf0, semA)
            off1, c3, c4 = do_chunk(2 * a + 1, pbuf1, qbuf1, semB)
            xcompute(2 * a, dbuf0)
            c1.wait(); c2.wait()
            ws0 = writes(off0, pbuf0, qbuf0, dbuf0)
            xcompute(2 * a + 1, dbuf1)
            c3.wait(); c4.wait()
            ws1 = writes(off1, pbuf1, qbuf1, dbuf1)
            for w in (*ws0, *ws1):
                w.wait()
            return carry

        lax.fori_loop(0, NCH // 2, pair_body, 0)
        off0, c1, c2 = do_chunk(NCH - 1, pbuf0, qbuf0, semA)
        xcompute(NCH - 1, dbuf0)
        c1.wait(); c2.wait()
        for w in writes(off0, pbuf0, qbuf0, dbuf0):
            w.wait()

    @functools.partial(
        pl.kernel,
        mesh=mesh,
        compiler_params=params,
        out_type=jax.ShapeDtypeStruct((NW, CAPP), jnp.int32),
        scratch_types=[
            pltpu.VMEM((SCN,), jnp.int32),
            pltpu.VMEM((SCN,), jnp.int32),
            pltpu.VMEM((CAPP,), jnp.int32),
            pltpu.SemaphoreType.DMA,
            pltpu.SemaphoreType.DMA,
        ],
    )
    def sc_scan(dst,
                packed_o,
                scan0, scan1, packed, semA, semB):
        c = lax.axis_index("c")
        s = lax.axis_index("s")
        wid = s * NC + c
        lo = wid * CR
        hi = lo + CR

        def process(scanbuf, ch, cnt):
            def vreg_body(g, cnt):
                for gg in (4 * g, 4 * g + 1, 4 * g + 2, 4 * g + 3):
                    v = scanbuf[pl.ds(gg * 16, 16)]
                    mask = (v >= lo) & (v < hi)
                    gid = ch * SCN + gg * 16 + iota16()
                    pk = gid | ((v - lo) << IDB)
                    plsc.store_compressed(packed.at[pl.ds(cnt, 16)], pk,
                                          mask=mask)
                    nm = plsc.all_reduce_population_count(mask)
                    cnt = cnt + nm[0]
                return cnt

            return lax.fori_loop(0, SCN // 64, vreg_body, cnt)

        def scan_pair(a, cnt):
            c0 = pltpu.async_copy(dst.at[pl.ds((2 * a) * SCN, SCN)],
                                  scan0, semA)
            c1 = pltpu.async_copy(dst.at[pl.ds((2 * a + 1) * SCN, SCN)],
                                  scan1, semB)
            c0.wait()
            cnt = process(scan0, 2 * a, cnt)
            c1.wait()
            cnt = process(scan1, 2 * a + 1, cnt)
            return cnt

        cnt = lax.fori_loop(0, NSC // 2, scan_pair, 0)

        def pad_body(g, carry):
            slot = g * 16 + iota16()
            pk = slot | (CR << IDB)
            plsc.store_scatter(packed, [slot], pk, mask=slot >= cnt)
            return carry

        lax.fori_loop(0, CAPP // 16, pad_body, 0)
        pltpu.sync_copy(packed, packed_o.at[wid])

    @functools.partial(
        pl.kernel,
        mesh=mesh,
        compiler_params=params,
        out_type=jax.ShapeDtypeStruct((NP * CW,), f32),
        scratch_types=[
            pltpu.VMEM((CAPP,), jnp.int32),
            pltpu.VMEM((KS,), jnp.int32),
            pltpu.VMEM((KS,), jnp.int32),
            pltpu.VMEM((KS,), jnp.int32),
            pltpu.VMEM((KS,), jnp.int32),
            pltpu.VMEM((KS, CW), f32),
            pltpu.VMEM((KS, CW), f32),
            pltpu.VMEM((ACC_R * CW,), f32),
            pltpu.SemaphoreType.DMA,
            pltpu.SemaphoreType.DMA,
        ],
    )
    def sc_scatter(comb, packed_i,
                   agg_o,
                   packed, ids0, ids1, lidx0, lidx1,
                   rb0, rb1, acc, semA, semB):
        c = lax.axis_index("c")
        s = lax.axis_index("s")
        wid = s * NC + c
        lo = wid * CR

        zp = pltpu.async_copy(packed_i.at[wid], packed, semB)

        def zero_body(r, carry):
            for jj in range(4):
                acc[pl.ds((r * 4 + jj) * 16, 16)] = jnp.zeros((16,), f32)
            return carry

        lax.fori_loop(0, ACC_R * CW // 64, zero_body, 0)
        npairs = CAPP // (2 * KS)
        zp.wait()

        def unpack(j, idsbuf, lidxbuf):
            for r in range(KS // 16):
                pv = packed[pl.ds(j * KS + r * 16, 16)]
                idsbuf[pl.ds(r * 16, 16)] = pv & ((1 << IDB) - 1)
                lidxbuf[pl.ds(r * 16, 16)] = pv >> IDB

        def accum(rowbuf, lidxbuf):
            def row_body(i2, carry2):
                for i in (2 * i2, 2 * i2 + 1):
                    lsplat = plsc.load_gather(
                        lidxbuf,
                        [jnp.broadcast_to(i, (16,)).astype(jnp.int32)])
                    base = lsplat * CW + iota16()
                    for jj in range((D + XP + 15) // 16):
                        v = rowbuf[i, pl.ds(jj * 16, 16)]
                        plsc.addupdate_scatter(acc, [base + jj * 16], v)
                return carry2

            lax.fori_loop(0, KS // 2, row_body, 0)

        def pair_body(a, carry):
            unpack(2 * a, ids0, lidx0)
            g0 = pltpu.async_copy(comb.at[ids0], rb0, semA)
            unpack(2 * a + 1, ids1, lidx1)
            g1 = pltpu.async_copy(comb.at[ids1], rb1, semB)
            g0.wait()
            accum(rb0, lidx0)
            g1.wait()
            accum(rb1, lidx1)
            return carry

        lax.fori_loop(0, npairs, pair_body, 0)

        pltpu.sync_copy(acc.at[pl.ds(0, CR * CW)],
                        agg_o.at[pl.ds(lo * CW, CR * CW)])

    return sc_gather, sc_scan, sc_scatter


def _edge_body(pd_ref, qs_ref, dd_ref, w2_ref, b2_ref, wc_ref,
               w1c_ref, comb_ref):
    dd = dd_ref[...]
    d2 = dd[:, 3:4]
    t1 = pd_ref[...] + qs_ref[...] + d2 * w1c_ref[...]
    a1 = t1 * jax.nn.sigmoid(t1)
    t2 = jnp.dot(a1, w2_ref[...], preferred_element_type=f32) + b2_ref[...]
    m = t2 * jax.nn.sigmoid(t2)
    cw = jnp.sum(m * wc_ref[...], axis=1, keepdims=True)
    comb_ref[:, :D] = m
    comb_ref[:, D:D + XP] = dd * cw


def _edge(pd, qs, dd, w2, b2, wc_row, w1c):
    be = 2000
    return pl.pallas_call(
        _edge_body,
        grid=(E // be,),
        in_specs=[
            pl.BlockSpec((be, D), lambda i: (i, 0)),
            pl.BlockSpec((be, D), lambda i: (i, 0)),
            pl.BlockSpec((be, XP), lambda i: (i, 0)),
            pl.BlockSpec((D, D), lambda i: (0, 0)),
            pl.BlockSpec((1, D), lambda i: (0, 0)),
            pl.BlockSpec((1, D), lambda i: (0, 0)),
            pl.BlockSpec((1, D), lambda i: (0, 0)),
        ],
        out_specs=pl.BlockSpec((be, CW), lambda i: (i, 0)),
        out_shape=jax.ShapeDtypeStruct((E, CW), f32),
    )(pd, qs, dd, w2, b2, wc_row, w1c)


def _node_body(h_ref, xp_ref, agg_ref, wu1a_ref, wu1b_ref, bu1_ref,
               wu2_ref, bu2_ref, xo_ref, ho_ref):
    agg = agg_ref[:, :D]
    aggx = agg_ref[:, D:D + XP]
    t = (jnp.dot(h_ref[...], wu1a_ref[...], preferred_element_type=f32)
         + jnp.dot(agg, wu1b_ref[...], preferred_element_type=f32)
         + bu1_ref[...])
    a = t * jax.nn.sigmoid(t)
    ho_ref[...] = (h_ref[...]
                   + jnp.dot(a, wu2_ref[...], preferred_element_type=f32)
                   + bu2_ref[...])
    xo_ref[...] = xp_ref[...] + aggx * (1.0 / DEG)


def _node(h, xp, agg, wu1a, wu1b, bu1, wu2, bu2):
    bn = 2000
    return pl.pallas_call(
        _node_body,
        grid=(N // bn,),
        in_specs=[
            pl.BlockSpec((bn, D), lambda i: (i, 0)),
            pl.BlockSpec((bn, XP), lambda i: (i, 0)),
            pl.BlockSpec((bn, CW), lambda i: (i, 0)),
            pl.BlockSpec((D, D), lambda i: (0, 0)),
            pl.BlockSpec((D, D), lambda i: (0, 0)),
            pl.BlockSpec((1, D), lambda i: (0, 0)),
            pl.BlockSpec((D, D), lambda i: (0, 0)),
            pl.BlockSpec((1, D), lambda i: (0, 0)),
        ],
        out_specs=[
            pl.BlockSpec((bn, XP), lambda i: (i, 0)),
            pl.BlockSpec((bn, D), lambda i: (i, 0)),
        ],
        out_shape=[
            jax.ShapeDtypeStruct((N, XP), f32),
            jax.ShapeDtypeStruct((N, D), f32),
        ],
    )(h, xp, agg, wu1a, wu1b, bu1, wu2, bu2)


def kernel(x, h, edge_index, W1, b1, W2, b2, Wc, Wu1, bu1, Wu2, bu2):
    src = edge_index[0].astype(jnp.int32)
    dst = edge_index[1].astype(jnp.int32)
    xp = jnp.pad(x.astype(f32), ((0, 0), (0, XP - x.shape[1])))
    w1a = W1[:D]
    w1b = W1[D:2 * D]
    w1c = W1[2 * D:2 * D + 1]
    b1r = b1.reshape(1, D)
    b2r = b2.reshape(1, D)
    wc_row = Wc.reshape(1, D)
    wu1a = Wu1[:D]
    wu1b = Wu1[D:2 * D]
    bu1r = bu1.reshape(1, D)
    bu2r = bu2.reshape(1, D)

    sc_gather, sc_scan, sc_scatter = _sc_kernels()
    p, q = _prep(h, w1a, w1b, b1r)
    packed = sc_scan(dst)
    pd, qs, dd = sc_gather(p, q, xp.reshape(-1), dst, src)
    dd = dd.reshape(E, XP)
    comb = _edge(pd, qs, dd, W2, b2r, wc_row, w1c)
    agg = sc_scatter(comb, packed).reshape(NP, CW)
    xo, ho = _node(h, xp, agg[:N], wu1a, wu1b, bu1r, Wu2, bu2r)
    return (xo[:, :x.shape[1]], ho)
